# baseline (speedup 1.0000x reference)
import functools

import jax
import jax.numpy as jnp
import numpy as np
from jax import lax
from jax.experimental import pallas as pl
from jax.experimental.pallas import tpu as pltpu
from jax.experimental.pallas import tpu_sc as plsc

N = 10000
D = 128
E = 320000
NC = 2
NS = 16
NW = NC * NS
EPT = E // NW
CH = 80
NCHUNK = EPT // CH

ROWB = 2000
GRID = N // ROWB


def _tc_first_body(x_ref, w_ref, asrc_ref, adst_ref, h_ref, asad_ref):
    h = jnp.dot(x_ref[...], w_ref[...], preferred_element_type=jnp.float32)
    h_ref[...] = h.astype(jnp.bfloat16)
    a_s = jnp.sum(h * asrc_ref[...], axis=1, keepdims=True)
    a_d = jnp.sum(h * adst_ref[...], axis=1, keepdims=True)
    asad_ref[...] = jnp.concatenate([a_s, a_d], axis=1)


def _tc_mid_body(acc_ref, den_ref, b_ref, w_ref, asrc_ref, adst_ref,
                 h_ref, asad_ref):
    num = acc_ref[0] + acc_ref[1]
    den = den_ref[0] + den_ref[1]
    x = num / (den + 1e-16) + b_ref[...]
    x = jnp.where(x >= 0.0, x, 0.1 * x)
    h = jnp.dot(x, w_ref[...], preferred_element_type=jnp.float32)
    h_ref[...] = h.astype(jnp.bfloat16)
    a_s = jnp.sum(h * asrc_ref[...], axis=1, keepdims=True)
    a_d = jnp.sum(h * adst_ref[...], axis=1, keepdims=True)
    asad_ref[...] = jnp.concatenate([a_s, a_d], axis=1)


def _tc_final_body(acc_ref, den_ref, b_ref, out_ref):
    num = acc_ref[0] + acc_ref[1]
    den = den_ref[0] + den_ref[1]
    x = num / (den + 1e-16) + b_ref[...]
    out_ref[...] = jnp.where(x >= 0.0, x, 0.1 * x)


_full = lambda *shape: pl.BlockSpec(shape, lambda i: (0,) * len(shape))


def _tc_first(x, W, asrc, adst):
    return pl.pallas_call(
        _tc_first_body,
        grid=(GRID,),
        in_specs=[
            pl.BlockSpec((ROWB, D), lambda i: (i, 0)),
            _full(D, D),
            _full(1, D),
            _full(1, D),
        ],
        out_specs=[
            pl.BlockSpec((ROWB, D), lambda i: (i, 0)),
            pl.BlockSpec((ROWB, 2), lambda i: (i, 0)),
        ],
        out_shape=[
            jax.ShapeDtypeStruct((N, D), jnp.bfloat16),
            jax.ShapeDtypeStruct((N, 2), jnp.float32),
        ],
    )(x, W, asrc, adst)


def _tc_mid(acc, den, b, W, asrc, adst):
    return pl.pallas_call(
        _tc_mid_body,
        grid=(GRID,),
        in_specs=[
            pl.BlockSpec((NC, ROWB, D), lambda i: (0, i, 0)),
            pl.BlockSpec((NC, ROWB, 1), lambda i: (0, i, 0)),
            _full(1, D),
            _full(D, D),
            _full(1, D),
            _full(1, D),
        ],
        out_specs=[
            pl.BlockSpec((ROWB, D), lambda i: (i, 0)),
            pl.BlockSpec((ROWB, 2), lambda i: (i, 0)),
        ],
        out_shape=[
            jax.ShapeDtypeStruct((N, D), jnp.bfloat16),
            jax.ShapeDtypeStruct((N, 2), jnp.float32),
        ],
    )(acc, den, b, W, asrc, adst)


def _tc_final(acc, den, b):
    return pl.pallas_call(
        _tc_final_body,
        grid=(GRID,),
        in_specs=[
            pl.BlockSpec((NC, ROWB, D), lambda i: (0, i, 0)),
            pl.BlockSpec((NC, ROWB, 1), lambda i: (0, i, 0)),
            _full(1, D),
        ],
        out_specs=pl.BlockSpec((ROWB, D), lambda i: (i, 0)),
        out_shape=jax.ShapeDtypeStruct((N, D), jnp.float32),
    )(acc, den, b)


_mesh = plsc.VectorSubcoreMesh(core_axis_name="c", subcore_axis_name="s")


@functools.partial(
    pl.kernel,
    out_type=(
        jax.ShapeDtypeStruct((NC, N, D), jnp.float32),
        jax.ShapeDtypeStruct((NC * N,), jnp.float32),
    ),
    mesh=_mesh,
    compiler_params=pltpu.CompilerParams(needs_layout_passes=False,
                                         use_tc_tiling_on_sc=False),
    scratch_types=[
        pltpu.VMEM((4, 2, CH), jnp.int32),
        pltpu.VMEM((2 * CH,), jnp.float32),
        pltpu.VMEM((2 * CH,), jnp.float32),
        pltpu.VMEM((2 * CH,), jnp.float32),
        pltpu.VMEM((CH, D // 2), jnp.int32),
        pltpu.VMEM((CH, D // 2), jnp.int32),
        pltpu.VMEM((CH, D), jnp.float32),
        pltpu.VMEM((CH, D), jnp.float32),
        pltpu.VMEM_SHARED((N, D), jnp.float32),
        pltpu.VMEM_SHARED((N,), jnp.float32),
        pltpu.SemaphoreType.DMA,
        pltpu.SemaphoreType.DMA,
        pltpu.SemaphoreType.DMA,
        pltpu.SemaphoreType.DMA,
        pltpu.SemaphoreType.DMA,
        pltpu.SemaphoreType.DMA,
        pltpu.SemaphoreType.DMA,
        pltpu.SemaphoreType.DMA,
        pltpu.SemaphoreType.DMA,
    ],
)
def _sc_edges(ei_hbm, as_hbm, ad_hbm, h_hbm, acc_out, den_out,
              eibb, wbb, asbb, adbb, rbf0, rbf1, rf0, rf1, acc_sh, den_sh,
              sem_g0, sem_g1, sem_s0, sem_s1,
              sem_e0, sem_e1, sem_e2, sem_e3, sem_wb):
    c = lax.axis_index("c")
    s = lax.axis_index("s")
    z16 = jnp.zeros((16,), jnp.float32)
    rbf = (rbf0, rbf1)
    rf = (rf0, rf1)
    sem_g = (sem_g0, sem_g1)
    sem_s = (sem_s0, sem_s1)
    sem_e = (sem_e0, sem_e1, sem_e2, sem_e3)

    def _zrow(i, _):
        for k in range(D // 16):
            rf0[i, pl.ds(k * 16, 16)] = z16
        return _
    lax.fori_loop(0, 40, _zrow, 0)

    for g in range(2 * CH // 16):
        wbb[pl.ds(g * 16, 16)] = z16

    @pl.when(s < 10)
    def _():
        zcps = [
            pltpu.async_copy(rf0.at[pl.ds(0, 40)],
                             acc_sh.at[pl.ds(s * 1000 + k * 40, 40)], sem_wb)
            for k in range(25)
        ]
        for cp in zcps:
            cp.wait()

    for m in range(8):
        kk = s + 16 * m
        @pl.when(kk < NCHUNK)
        def _zd():
            pltpu.sync_copy(wbb.at[pl.ds(0, CH)],
                            den_sh.at[pl.ds(kk * CH, CH)])

    plsc.subcore_barrier()

    cbase = (c * NS + s) * NCHUNK

    def _compute(b, rb):
        for g in range(CH // 16):
            av = asbb[pl.ds(rb * CH + g * 16, 16)]
            dv = adbb[pl.ds(rb * CH + g * 16, 16)]
            e = av + dv
            e = jnp.where(e >= 0.0, e, 0.2 * e)
            wbb[pl.ds(rb * CH + g * 16, 16)] = jnp.exp(e)

        @plsc.parallel_loop(0, CH, 1, unroll=4)
        def _(i):
            ws = plsc.load_gather(wbb, [jnp.full((16,), rb * CH, jnp.int32) + i])
            for k in range(D // 32):
                x16 = rbf[rb][i, pl.ds(k * 16, 16)]
                x32 = plsc.bitcast(x16, jnp.bfloat16)
                lo, hi = plsc.unpack(x32, format=plsc.PackFormat.INTERLEAVED)
                rf[rb][i, pl.ds(k * 32, 16)] = lo * ws
                rf[rb][i, pl.ds(k * 32 + 16, 16)] = hi * ws

    def _issue_gathers(e, rb):
        pltpu.async_copy(h_hbm.at[eibb.at[e, 0]], rbf[rb], sem_g[rb])
        pltpu.async_copy(as_hbm.at[eibb.at[e, 0]],
                         asbb.at[pl.ds(rb * CH, CH)], sem_g[rb])
        pltpu.async_copy(ad_hbm.at[eibb.at[e, 1]],
                         adbb.at[pl.ds(rb * CH, CH)], sem_g[rb])

    def _issue_scatter(b, rb):
        pltpu.async_copy(rf[rb], acc_sh.at[eibb.at[b, 1]], sem_s[rb], add=True)
        pltpu.async_copy(wbb.at[pl.ds(rb * CH, CH)], den_sh.at[eibb.at[b, 1]],
                         sem_s[rb], add=True)

    def _wait_scatter(b):
        pltpu.make_async_copy(acc_out.at[0, pl.ds(0, CH)], rf[b],
                              sem_s[b]).wait()
        pltpu.make_async_copy(den_out.at[pl.ds(0, CH)],
                              wbb.at[pl.ds(b * CH, CH)], sem_s[b]).wait()

    def _wait_gather(b):
        pltpu.make_async_copy(h_hbm.at[pl.ds(0, CH)], rbf[b], sem_g[b]).wait()
        pltpu.make_async_copy(as_hbm.at[pl.ds(0, CH)],
                              asbb.at[pl.ds(b * CH, CH)], sem_g[b]).wait()
        pltpu.make_async_copy(ad_hbm.at[pl.ds(0, CH)],
                              adbb.at[pl.ds(b * CH, CH)], sem_g[b]).wait()

    def _wait_ei(e):
        pltpu.make_async_copy(ei_hbm.at[cbase], eibb.at[e], sem_e[e]).wait()

    pltpu.sync_copy(ei_hbm.at[cbase], eibb.at[0])
    _issue_gathers(0, 0)
    pltpu.async_copy(ei_hbm.at[cbase + 1], eibb.at[1], sem_e1)

    def _quad(q, carry):
        for b in range(4):
            j = 4 * q + b
            rb = b % 2
            _wait_gather(rb)
            if b == 0:
                @pl.when(q > 0)
                def _ws():
                    _wait_scatter(1)
            else:
                _wait_scatter(1 - rb)
            _wait_ei((b + 1) % 4)
            _issue_gathers((b + 1) % 4, 1 - rb)
            @pl.when(j + 2 < NCHUNK)
            def _pe():
                pltpu.async_copy(ei_hbm.at[cbase + j + 2],
                                 eibb.at[(b + 2) % 4], sem_e[(b + 2) % 4])
            _compute(b, rb)
            _issue_scatter(b, rb)
        return carry

    lax.fori_loop(0, NCHUNK // 4, _quad, 0)

    _wait_gather(0)
    _wait_scatter(1)
    _compute(0, 0)
    _issue_scatter(0, 0)
    _wait_scatter(0)

    plsc.subcore_barrier()

    @pl.when(s < 10)
    def _():
        prev = [None, None]
        for k in range(25):
            b = k % 2
            if prev[b] is not None:
                prev[b].wait()
            pltpu.sync_copy(acc_sh.at[pl.ds(s * 1000 + k * 40, 40)],
                            rf[b].at[pl.ds(0, 40)])
            prev[b] = pltpu.async_copy(
                rf[b].at[pl.ds(0, 40)],
                acc_out.at[c, pl.ds(s * 1000 + k * 40, 40)], sem_wb)
        prev[0].wait()
        prev[1].wait()
    for m in range(8):
        kk = s + 16 * m
        @pl.when(kk < NCHUNK)
        def _dwb():
            pltpu.sync_copy(den_sh.at[pl.ds(kk * CH, CH)],
                            wbb.at[pl.ds(0, CH)])
            pltpu.sync_copy(wbb.at[pl.ds(0, CH)],
                            den_out.at[pl.ds(c * N + kk * CH, CH)])


_TRUECOL = np.concatenate(
    [np.concatenate([32 * k + 2 * np.arange(16),
                     32 * k + 2 * np.arange(16) + 1]) for k in range(D // 32)])
_INVCOL = np.argsort(_TRUECOL)


def kernel(x, edge_index, W0, a_src0, a_dst0, b0, W1, a_src1, a_dst1, b1,
           W2, a_src2, a_dst2, b2):
    ei = edge_index.astype(jnp.int32).reshape(2, NW * NCHUNK, CH)
    ei = ei.transpose(1, 0, 2)

    r = lambda v: v.reshape(1, D)

    v32 = lambda hb: jax.lax.bitcast_convert_type(
        hb.reshape(N, D // 2, 2), jnp.int32)

    h, asad = _tc_first(x, W0, r(a_src0), r(a_dst0))
    acc, den = _sc_edges(ei, asad[:, 0], asad[:, 1], v32(h))

    h, asad = _tc_mid(acc, den.reshape(NC, N, 1), r(b0[_TRUECOL]),
                      W1[_TRUECOL], r(a_src1), r(a_dst1))
    acc, den = _sc_edges(ei, asad[:, 0], asad[:, 1], v32(h))

    h, asad = _tc_mid(acc, den.reshape(NC, N, 1), r(b1[_TRUECOL]),
                      W2[_TRUECOL], r(a_src2), r(a_dst2))
    acc, den = _sc_edges(ei, asad[:, 0], asad[:, 1], v32(h))

    out = _tc_final(acc, den.reshape(NC, N, 1), r(b2[_TRUECOL]))
    return out[:, _INVCOL]


# speedup vs baseline: 1.1795x; 1.1795x over previous
import functools

import jax
import jax.numpy as jnp
from jax import lax
from jax.experimental import pallas as pl
from jax.experimental.pallas import tpu as pltpu
from jax.experimental.pallas import tpu_sc as plsc

N = 10000
D = 128
E = 320000
NC = 2
NS = 16
NW = NC * NS
EPT = E // NW
CH = 80
NCHUNK = EPT // CH

ROWB = 2000
GRID = N // ROWB


def _tc_first_body(x_ref, w_ref, asrc_ref, adst_ref, h_ref, asad_ref):
    h = jnp.dot(x_ref[...], w_ref[...], preferred_element_type=jnp.float32)
    h_ref[...] = h
    a_s = jnp.sum(h * asrc_ref[...], axis=1, keepdims=True)
    a_d = jnp.sum(h * adst_ref[...], axis=1, keepdims=True)
    asad_ref[...] = jnp.concatenate([a_s, a_d], axis=1)


def _tc_mid_body(acc_ref, den_ref, b_ref, w_ref, asrc_ref, adst_ref,
                 h_ref, asad_ref):
    num = acc_ref[0] + acc_ref[1]
    den = den_ref[0] + den_ref[1]
    x = num / (den + 1e-16) + b_ref[...]
    x = jnp.where(x >= 0.0, x, 0.1 * x)
    h = jnp.dot(x, w_ref[...], preferred_element_type=jnp.float32)
    h_ref[...] = h
    a_s = jnp.sum(h * asrc_ref[...], axis=1, keepdims=True)
    a_d = jnp.sum(h * adst_ref[...], axis=1, keepdims=True)
    asad_ref[...] = jnp.concatenate([a_s, a_d], axis=1)


def _tc_final_body(acc_ref, den_ref, b_ref, out_ref):
    num = acc_ref[0] + acc_ref[1]
    den = den_ref[0] + den_ref[1]
    x = num / (den + 1e-16) + b_ref[...]
    out_ref[...] = jnp.where(x >= 0.0, x, 0.1 * x)


_full = lambda *shape: pl.BlockSpec(shape, lambda i: (0,) * len(shape))


def _tc_first(x, W, asrc, adst):
    return pl.pallas_call(
        _tc_first_body,
        grid=(GRID,),
        in_specs=[
            pl.BlockSpec((ROWB, D), lambda i: (i, 0)),
            _full(D, D),
            _full(1, D),
            _full(1, D),
        ],
        out_specs=[
            pl.BlockSpec((ROWB, D), lambda i: (i, 0)),
            pl.BlockSpec((ROWB, 2), lambda i: (i, 0)),
        ],
        out_shape=[
            jax.ShapeDtypeStruct((N, D), jnp.float32),
            jax.ShapeDtypeStruct((N, 2), jnp.float32),
        ],
    )(x, W, asrc, adst)


def _tc_mid(acc, den, b, W, asrc, adst):
    return pl.pallas_call(
        _tc_mid_body,
        grid=(GRID,),
        in_specs=[
            pl.BlockSpec((NC, ROWB, D), lambda i: (0, i, 0)),
            pl.BlockSpec((NC, ROWB, 1), lambda i: (0, i, 0)),
            _full(1, D),
            _full(D, D),
            _full(1, D),
            _full(1, D),
        ],
        out_specs=[
            pl.BlockSpec((ROWB, D), lambda i: (i, 0)),
            pl.BlockSpec((ROWB, 2), lambda i: (i, 0)),
        ],
        out_shape=[
            jax.ShapeDtypeStruct((N, D), jnp.float32),
            jax.ShapeDtypeStruct((N, 2), jnp.float32),
        ],
    )(acc, den, b, W, asrc, adst)


def _tc_final(acc, den, b):
    return pl.pallas_call(
        _tc_final_body,
        grid=(GRID,),
        in_specs=[
            pl.BlockSpec((NC, ROWB, D), lambda i: (0, i, 0)),
            pl.BlockSpec((NC, ROWB, 1), lambda i: (0, i, 0)),
            _full(1, D),
        ],
        out_specs=pl.BlockSpec((ROWB, D), lambda i: (i, 0)),
        out_shape=jax.ShapeDtypeStruct((N, D), jnp.float32),
    )(acc, den, b)


_mesh = plsc.VectorSubcoreMesh(core_axis_name="c", subcore_axis_name="s")


@functools.partial(
    pl.kernel,
    out_type=(
        jax.ShapeDtypeStruct((NC, N, D), jnp.float32),
        jax.ShapeDtypeStruct((NC * N,), jnp.float32),
    ),
    mesh=_mesh,
    compiler_params=pltpu.CompilerParams(needs_layout_passes=False),
    scratch_types=[
        pltpu.VMEM((2 * N,), jnp.float32),
        pltpu.VMEM((2, CH), jnp.int32),
        pltpu.VMEM((2, CH), jnp.int32),
        pltpu.VMEM((2, CH), jnp.int32),
        pltpu.VMEM((2, CH), jnp.int32),
        pltpu.VMEM((CH,), jnp.float32),
        pltpu.VMEM((CH,), jnp.float32),
        pltpu.VMEM((CH, D), jnp.float32),
        pltpu.VMEM((CH, D), jnp.float32),
        pltpu.VMEM((1000,), jnp.float32),
        pltpu.VMEM_SHARED((N, D), jnp.float32),
        pltpu.VMEM_SHARED((N,), jnp.float32),
        pltpu.SemaphoreType.DMA,
        pltpu.SemaphoreType.DMA,
        pltpu.SemaphoreType.DMA,
        pltpu.SemaphoreType.DMA,
        pltpu.SemaphoreType.DMA,
        pltpu.SemaphoreType.DMA,
        pltpu.SemaphoreType.DMA,
        pltpu.SemaphoreType.DMA,
        pltpu.SemaphoreType.DMA,
    ],
)
def _sc_edges(ei_hbm, asad_hbm, h_hbm, acc_out, den_out,
              asad_v, eib0, eib1, eib2, eib3, wb0, wb1, rowsb0, rowsb1,
              zden_v, acc_sh, den_sh,
              sem_g0, sem_g1, sem_s0, sem_s1,
              sem_e0, sem_e1, sem_e2, sem_e3, sem_wb):
    c = lax.axis_index("c")
    s = lax.axis_index("s")
    z16 = jnp.zeros((16,), jnp.float32)
    eib = (eib0, eib1, eib2, eib3)
    wb = (wb0, wb1)
    rows = (rowsb0, rowsb1)
    sem_g = (sem_g0, sem_g1)
    sem_s = (sem_s0, sem_s1)
    sem_e = (sem_e0, sem_e1, sem_e2, sem_e3)

    def _zrow(i, _):
        for k in range(D // 16):
            rowsb0[i, pl.ds(k * 16, 16)] = z16
        return _
    lax.fori_loop(0, 40, _zrow, 0)

    def _zden(i, _):
        zden_v[pl.ds(i * 16, 16)] = z16
        return _
    lax.fori_loop(0, 62, _zden, 0)
    zden_v[pl.ds(984, 16)] = z16

    @pl.when(s < 10)
    def _():
        zcps = [
            pltpu.async_copy(rowsb0.at[pl.ds(0, 40)],
                             acc_sh.at[pl.ds(s * 1000 + k * 40, 40)], sem_wb)
            for k in range(25)
        ]
        for cp in zcps:
            cp.wait()
        pltpu.sync_copy(zden_v, den_sh.at[pl.ds(s * 1000, 1000)])

    pltpu.sync_copy(asad_hbm, asad_v)

    plsc.subcore_barrier()

    cbase = (c * NS + s) * NCHUNK

    def _compute(b, rb):
        for g in range(CH // 16):
            sidx = eib[b][0, pl.ds(g * 16, 16)]
            didx = eib[b][1, pl.ds(g * 16, 16)]
            av = plsc.load_gather(asad_v, [sidx * 2])
            dv = plsc.load_gather(asad_v, [didx * 2 + 1])
            e = av + dv
            e = jnp.where(e >= 0.0, e, 0.2 * e)
            wb[rb][pl.ds(g * 16, 16)] = jnp.exp(e)

        @plsc.parallel_loop(0, CH, 1, unroll=8)
        def _(i):
            ws = plsc.load_gather(wb[rb], [jnp.full((16,), i, jnp.int32)])
            for k in range(D // 16):
                rows[rb][i, pl.ds(k * 16, 16)] = (
                    rows[rb][i, pl.ds(k * 16, 16)] * ws)

    def _issue_scatter(b, rb):
        pltpu.async_copy(rows[rb], acc_sh.at[eib[b].at[1]], sem_s[rb], add=True)
        pltpu.async_copy(wb[rb], den_sh.at[eib[b].at[1]], sem_s[rb], add=True)

    def _wait_scatter(b):
        pltpu.make_async_copy(h_hbm.at[pl.ds(0, CH)], rows[b], sem_s[b]).wait()
        pltpu.make_async_copy(den_out.at[pl.ds(0, CH)], wb[b], sem_s[b]).wait()

    def _wait_gather(b):
        pltpu.make_async_copy(h_hbm.at[pl.ds(0, CH)], rows[b], sem_g[b]).wait()

    def _wait_ei(e):
        pltpu.make_async_copy(ei_hbm.at[cbase], eib[e], sem_e[e]).wait()

    pltpu.sync_copy(ei_hbm.at[cbase], eib0)
    pltpu.async_copy(h_hbm.at[eib0.at[0]], rowsb0, sem_g0)
    pltpu.async_copy(ei_hbm.at[cbase + 1], eib1, sem_e1)

    def _quad(q, carry):
        for b in range(4):
            j = 4 * q + b
            rb = b % 2
            _wait_gather(rb)
            if b == 0:
                @pl.when(q > 0)
                def _ws():
                    _wait_scatter(1)
            else:
                _wait_scatter(1 - rb)
            _wait_ei((b + 1) % 4)
            pltpu.async_copy(h_hbm.at[eib[(b + 1) % 4].at[0]],
                             rows[1 - rb], sem_g[1 - rb])
            @pl.when(j + 2 < NCHUNK)
            def _pe():
                pltpu.async_copy(ei_hbm.at[cbase + j + 2],
                                 eib[(b + 2) % 4], sem_e[(b + 2) % 4])
            _compute(b, rb)
            _issue_scatter(b, rb)
        return carry

    lax.fori_loop(0, NCHUNK // 4, _quad, 0)

    _wait_gather(0)
    _wait_scatter(1)
    _compute(0, 0)
    _issue_scatter(0, 0)
    _wait_scatter(0)

    plsc.subcore_barrier()

    @pl.when(s < 10)
    def _():
        prev = [None, None]
        for k in range(25):
            b = k % 2
            if prev[b] is not None:
                prev[b].wait()
            pltpu.sync_copy(acc_sh.at[pl.ds(s * 1000 + k * 40, 40)],
                            rows[b].at[pl.ds(0, 40)])
            prev[b] = pltpu.async_copy(
                rows[b].at[pl.ds(0, 40)],
                acc_out.at[c, pl.ds(s * 1000 + k * 40, 40)], sem_wb)
        prev[0].wait()
        prev[1].wait()
        pltpu.sync_copy(den_sh.at[pl.ds(s * 1000, 1000)], zden_v)
        pltpu.sync_copy(zden_v, den_out.at[pl.ds(c * N + s * 1000, 1000)])


def kernel(x, edge_index, W0, a_src0, a_dst0, b0, W1, a_src1, a_dst1, b1,
           W2, a_src2, a_dst2, b2):
    ei = edge_index.astype(jnp.int32).reshape(2, NW * NCHUNK, CH)
    ei = ei.transpose(1, 0, 2)

    r = lambda v: v.reshape(1, D)

    h, asad = _tc_first(x, W0, r(a_src0), r(a_dst0))
    acc, den = _sc_edges(ei, asad.reshape(2 * N), h)

    h, asad = _tc_mid(acc, den.reshape(NC, N, 1), r(b0), W1, r(a_src1), r(a_dst1))
    acc, den = _sc_edges(ei, asad.reshape(2 * N), h)

    h, asad = _tc_mid(acc, den.reshape(NC, N, 1), r(b1), W2, r(a_src2), r(a_dst2))
    acc, den = _sc_edges(ei, asad.reshape(2 * N), h)

    return _tc_final(acc, den.reshape(NC, N, 1), r(b2))
